# Initial kernel scaffold; baseline (speedup 1.0000x reference)
import jax
import jax.numpy as jnp
from jax.experimental import pallas as pl


def kernel(x, edge_index_0, weights_0, bias_0, edge_index_1, weights_1, bias_1, edge_index_2, weights_2, bias_2):
    raise NotImplementedError("write your pallas kernel here")


# speedup vs baseline: 6.1982x; 6.1982x over previous
import functools

import jax
import jax.numpy as jnp
from jax import lax
from jax.experimental import pallas as pl
from jax.experimental.pallas import tpu as pltpu
from jax.experimental.pallas import tpu_sc as plsc

_SIZES = (512, 4096, 4096, 256)
_DEG = 16
_B = 256
_L = 16
_NBLK = _B // _L


@functools.lru_cache(maxsize=None)
def _layer(prev: int, curr: int):
    cph = curr // 2
    cd = min(cph, 512)
    nch = cph // cd
    ng = cd // _L
    mesh = plsc.VectorSubcoreMesh(core_axis_name="c", subcore_axis_name="s")

    @functools.partial(
        pl.kernel,
        out_type=jax.ShapeDtypeStruct((_NBLK, _L, curr), jnp.float32),
        mesh=mesh,
        compiler_params=pltpu.CompilerParams(needs_layout_passes=False),
        scratch_types=[
            pltpu.VMEM((_L, prev), jnp.float32),
            pltpu.VMEM((_DEG, cd), jnp.int32),
            pltpu.VMEM((_DEG, cd), jnp.float32),
            pltpu.VMEM((cd,), jnp.float32),
            pltpu.VMEM((_L, cd), jnp.float32),
        ],
    )
    def layer_k(act_hbm, src_hbm, w_hbm, b_hbm, out_hbm, table_v, idx_v, w_v, b_v, out_v):
        blk = lax.axis_index("s")
        half = lax.axis_index("c")
        pltpu.sync_copy(act_hbm.at[blk], table_v)
        for ci in range(nch):
            d0 = half * cph + ci * cd
            pltpu.sync_copy(src_hbm.at[:, pl.ds(d0, cd)], idx_v)
            pltpu.sync_copy(w_hbm.at[:, pl.ds(d0, cd)], w_v)
            pltpu.sync_copy(b_hbm.at[pl.ds(d0, cd)], b_v)

            def group(g, _):
                col0 = pl.multiple_of(g * _L, _L)
                bias_vec = b_v[pl.ds(col0, _L)]
                icols = [idx_v[k, pl.ds(col0, _L)] for k in range(_DEG)]
                wcols = [w_v[k, pl.ds(col0, _L)] for k in range(_DEG)]
                for j in range(_L):
                    jvec = jnp.full((_L,), j, jnp.int32)
                    acc = bias_vec
                    for k in range(_DEG):
                        rows = plsc.load_gather(table_v, [jvec, icols[k]])
                        acc = acc + wcols[k] * rows
                    out_v[j, pl.ds(col0, _L)] = jnp.maximum(acc, 0.0)
                return 0

            lax.fori_loop(0, ng, group, 0)
            pltpu.sync_copy(out_v, out_hbm.at[blk, :, pl.ds(d0, cd)])

    return layer_k


def kernel(x, edge_index_0, weights_0, bias_0, edge_index_1, weights_1, bias_1,
           edge_index_2, weights_2, bias_2):
    a = x.reshape(_NBLK, _L, _SIZES[0])
    params = ((edge_index_0, weights_0, bias_0),
              (edge_index_1, weights_1, bias_1),
              (edge_index_2, weights_2, bias_2))
    for li, (ei, w, b) in enumerate(params):
        prev, curr = _SIZES[li], _SIZES[li + 1]
        src_cols = ei[1].astype(jnp.int32).reshape(curr, _DEG).T
        w_cols = w.reshape(curr, _DEG).T
        a = _layer(prev, curr)(a, src_cols, w_cols, b)
    return a.reshape(_B, _SIZES[-1])


# speedup vs baseline: 8.0502x; 1.2988x over previous
import functools

import jax
import jax.numpy as jnp
from jax import lax
from jax.experimental import pallas as pl
from jax.experimental.pallas import tpu as pltpu
from jax.experimental.pallas import tpu_sc as plsc

_SIZES = (512, 4096, 4096, 256)
_DEG = 16
_B = 256
_L = 16
_NBLK = _B // _L


@functools.lru_cache(maxsize=None)
def _layer(prev: int, curr: int):
    cph = curr // 2
    cd = min(cph, 512)
    nch = cph // cd
    ng = cd // _L
    nbuf = min(2, nch)
    mesh = plsc.VectorSubcoreMesh(core_axis_name="c", subcore_axis_name="s")

    scratch = [pltpu.VMEM((_L, prev), jnp.float32)]
    for _ in range(nbuf):
        scratch += [
            pltpu.VMEM((_DEG, cd), jnp.int32),
            pltpu.VMEM((_DEG, cd), jnp.float32),
            pltpu.VMEM((cd,), jnp.float32),
            pltpu.VMEM((_L, cd), jnp.float32),
        ]
    scratch += [pltpu.SemaphoreType.DMA] * (1 + 2 * nbuf)

    @functools.partial(
        pl.kernel,
        out_type=jax.ShapeDtypeStruct((_NBLK, _L, curr), jnp.float32),
        mesh=mesh,
        compiler_params=pltpu.CompilerParams(needs_layout_passes=False),
        scratch_types=scratch,
    )
    def layer_k(act_hbm, src_hbm, w_hbm, b_hbm, out_hbm, table_v, *rest):
        bufs = [rest[4 * i:4 * i + 4] for i in range(nbuf)]
        sem_t = rest[4 * nbuf]
        sem_in = rest[4 * nbuf + 1:4 * nbuf + 1 + nbuf]
        sem_out = rest[4 * nbuf + 1 + nbuf:]
        blk = lax.axis_index("s")
        half = lax.axis_index("c")
        d_of = lambda ci: half * cph + ci * cd

        tbl_cp = pltpu.async_copy(act_hbm.at[blk], table_v, sem_t)

        def start_inputs(ci):
            buf = ci % nbuf
            ib, wb, bb, _ = bufs[buf]
            d0 = d_of(ci)
            return (
                pltpu.async_copy(src_hbm.at[:, pl.ds(d0, cd)], ib, sem_in[buf]),
                pltpu.async_copy(w_hbm.at[:, pl.ds(d0, cd)], wb, sem_in[buf]),
                pltpu.async_copy(b_hbm.at[pl.ds(d0, cd)], bb, sem_in[buf]),
            )

        pending_in = start_inputs(0)
        tbl_cp.wait()
        pending_out = [None] * nbuf
        for ci in range(nch):
            buf = ci % nbuf
            for h in pending_in:
                h.wait()
            if ci + 1 < nch:
                pending_in = start_inputs(ci + 1)
            ib, wb, bb, ob = bufs[buf]
            if pending_out[buf] is not None:
                pending_out[buf].wait()

            def group(g, _, ib=ib, wb=wb, bb=bb, ob=ob):
                col0 = pl.multiple_of(g * _L, _L)
                bias_vec = bb[pl.ds(col0, _L)]
                icols = [ib[k, pl.ds(col0, _L)] for k in range(_DEG)]
                wcols = [wb[k, pl.ds(col0, _L)] for k in range(_DEG)]
                for j in range(_L):
                    jvec = jnp.full((_L,), j, jnp.int32)
                    terms = [wcols[k] * plsc.load_gather(table_v, [jvec, icols[k]])
                             for k in range(_DEG)]
                    while len(terms) > 1:
                        terms = [terms[i] + terms[i + 1]
                                 for i in range(0, len(terms), 2)]
                    ob[j, pl.ds(col0, _L)] = jnp.maximum(bias_vec + terms[0], 0.0)
                return 0

            lax.fori_loop(0, ng, group, 0)
            pending_out[buf] = pltpu.async_copy(
                ob, out_hbm.at[blk, :, pl.ds(d_of(ci), cd)], sem_out[buf])
        for h in pending_out:
            if h is not None:
                h.wait()

    return layer_k


def kernel(x, edge_index_0, weights_0, bias_0, edge_index_1, weights_1, bias_1,
           edge_index_2, weights_2, bias_2):
    a = x.reshape(_NBLK, _L, _SIZES[0])
    params = ((edge_index_0, weights_0, bias_0),
              (edge_index_1, weights_1, bias_1),
              (edge_index_2, weights_2, bias_2))
    for li, (ei, w, b) in enumerate(params):
        prev, curr = _SIZES[li], _SIZES[li + 1]
        src_cols = ei[1].astype(jnp.int32).reshape(curr, _DEG).T
        w_cols = w.reshape(curr, _DEG).T
        a = _layer(prev, curr)(a, src_cols, w_cols, b)
    return a.reshape(_B, _SIZES[-1])


# speedup vs baseline: 9.1213x; 1.1331x over previous
import functools

import jax
import jax.numpy as jnp
from jax import lax
from jax.experimental import pallas as pl
from jax.experimental.pallas import tpu as pltpu
from jax.experimental.pallas import tpu_sc as plsc

_SIZES = (512, 4096, 4096, 256)
_DEG = 16
_B = 256
_L = 16
_NBLK = _B // _L


@functools.lru_cache(maxsize=None)
def _layer(prev: int, curr: int):
    cph = curr // 2
    cd = min(cph, 512)
    nch = cph // cd
    ng = cd // _L
    nbuf = min(2, nch)
    mesh = plsc.VectorSubcoreMesh(core_axis_name="c", subcore_axis_name="s")

    scratch = [pltpu.VMEM((_L, prev), jnp.float32)]
    for _ in range(nbuf):
        scratch += [
            pltpu.VMEM((_DEG, cd), jnp.int32),
            pltpu.VMEM((_DEG, cd), jnp.float32),
            pltpu.VMEM((cd,), jnp.float32),
            pltpu.VMEM((_L, cd), jnp.float32),
        ]
    scratch += [pltpu.SemaphoreType.DMA] * (1 + 2 * nbuf)

    @functools.partial(
        pl.kernel,
        out_type=jax.ShapeDtypeStruct((_NBLK, _L, curr), jnp.float32),
        mesh=mesh,
        compiler_params=pltpu.CompilerParams(needs_layout_passes=False),
        scratch_types=scratch,
    )
    def layer_k(act_hbm, src_hbm, w_hbm, b_hbm, out_hbm, table_v, *rest):
        bufs = [rest[4 * i:4 * i + 4] for i in range(nbuf)]
        sem_t = rest[4 * nbuf]
        sem_in = rest[4 * nbuf + 1:4 * nbuf + 1 + nbuf]
        sem_out = rest[4 * nbuf + 1 + nbuf:]
        blk = lax.axis_index("s")
        half = lax.axis_index("c")
        d_of = lambda ci: half * cph + ci * cd

        tbl_cp = pltpu.async_copy(act_hbm.at[blk], table_v, sem_t)

        def start_inputs(ci):
            buf = ci % nbuf
            ib, wb, bb, _ = bufs[buf]
            d0 = d_of(ci)
            return (
                pltpu.async_copy(src_hbm.at[:, pl.ds(d0, cd)], ib, sem_in[buf]),
                pltpu.async_copy(w_hbm.at[:, pl.ds(d0, cd)], wb, sem_in[buf]),
                pltpu.async_copy(b_hbm.at[pl.ds(d0, cd)], bb, sem_in[buf]),
            )

        pending_in = start_inputs(0)
        tbl_cp.wait()
        pending_out = [None] * nbuf
        for ci in range(nch):
            buf = ci % nbuf
            for h in pending_in:
                h.wait()
            if ci + 1 < nch:
                pending_in = start_inputs(ci + 1)
            ib, wb, bb, ob = bufs[buf]
            if pending_out[buf] is not None:
                pending_out[buf].wait()

            def group(g, _, ib=ib, wb=wb, bb=bb, ob=ob):
                col0 = pl.multiple_of(g * _L, _L)
                bias_vec = bb[pl.ds(col0, _L)]
                accs = [bias_vec] * _L
                for k in range(_DEG):
                    icol = ib[k, pl.ds(col0, _L)]
                    wcol = wb[k, pl.ds(col0, _L)]
                    for j in range(_L):
                        jvec = jnp.full((_L,), j, jnp.int32)
                        accs[j] = accs[j] + wcol * plsc.load_gather(
                            table_v, [jvec, icol])
                for j in range(_L):
                    ob[j, pl.ds(col0, _L)] = jnp.maximum(accs[j], 0.0)
                return 0

            lax.fori_loop(0, ng, group, 0)
            pending_out[buf] = pltpu.async_copy(
                ob, out_hbm.at[blk, :, pl.ds(d_of(ci), cd)], sem_out[buf])
        for h in pending_out:
            if h is not None:
                h.wait()

    return layer_k


def kernel(x, edge_index_0, weights_0, bias_0, edge_index_1, weights_1, bias_1,
           edge_index_2, weights_2, bias_2):
    a = x.reshape(_NBLK, _L, _SIZES[0])
    params = ((edge_index_0, weights_0, bias_0),
              (edge_index_1, weights_1, bias_1),
              (edge_index_2, weights_2, bias_2))
    for li, (ei, w, b) in enumerate(params):
        prev, curr = _SIZES[li], _SIZES[li + 1]
        src_cols = ei[1].astype(jnp.int32).reshape(curr, _DEG).T
        w_cols = w.reshape(curr, _DEG).T
        a = _layer(prev, curr)(a, src_cols, w_cols, b)
    return a.reshape(_B, _SIZES[-1])


# speedup vs baseline: 9.2931x; 1.0188x over previous
import functools

import jax
import jax.numpy as jnp
from jax import lax
from jax.experimental import pallas as pl
from jax.experimental.pallas import tpu as pltpu
from jax.experimental.pallas import tpu_sc as plsc

_SIZES = (512, 4096, 4096, 256)
_DEG = 16
_B = 256
_L = 16
_NBLK = _B // _L
_PMAX = 4096
_CDMAX = 256
_NBUF = 2


def _fused():
    mesh = plsc.VectorSubcoreMesh(core_axis_name="c", subcore_axis_name="s")

    scratch = [pltpu.VMEM((_L, _PMAX), jnp.float32)]
    for _ in range(_NBUF):
        scratch += [
            pltpu.VMEM((_DEG, _CDMAX), jnp.int32),
            pltpu.VMEM((_DEG, _CDMAX), jnp.float32),
            pltpu.VMEM((_CDMAX,), jnp.float32),
            pltpu.VMEM((_L, _CDMAX), jnp.float32),
        ]
    scratch += [
        pltpu.VMEM_SHARED((8, _L, _SIZES[1]), jnp.float32),
    ]
    scratch += [pltpu.SemaphoreType.DMA] * (1 + 2 * _NBUF)

    @functools.partial(
        pl.kernel,
        out_type=(jax.ShapeDtypeStruct((_NBLK, _L, _SIZES[3]), jnp.float32),
                  jax.ShapeDtypeStruct((_NBLK, _L, _SIZES[2]), jnp.float32)),
        mesh=mesh,
        compiler_params=pltpu.CompilerParams(needs_layout_passes=False),
        scratch_types=scratch,
    )
    def fused_k(x_hbm, s0_hbm, w0_hbm, b0_hbm, s1_hbm, w1_hbm, b1_hbm,
                s2_hbm, w2_hbm, b2_hbm, out_hbm, act_b, table_v, *rest):
        bufs = [rest[4 * i:4 * i + 4] for i in range(_NBUF)]
        act_a = rest[4 * _NBUF]
        sem_t = rest[4 * _NBUF + 1]
        sem_in = rest[4 * _NBUF + 2:4 * _NBUF + 2 + _NBUF]
        sem_out = rest[4 * _NBUF + 2 + _NBUF:]

        s = lax.axis_index("s")
        c = lax.axis_index("c")
        blk = c * 8 + s // 2
        sblk = s // 2
        half = s % 2

        def run_layer(prev, curr, table_src, src_hbm, w_hbm, b_hbm, out_dst):
            cph = curr // 2
            cd = min(cph, _CDMAX)
            nch = cph // cd
            ng = cd // _L
            d_of = lambda ci: half * cph + ci * cd

            tbl_cp = pltpu.async_copy(
                table_src, table_v.at[:, pl.ds(0, prev)], sem_t)

            def start_inputs(ci):
                buf = ci % _NBUF
                ib, wb, bb, _ = bufs[buf]
                d0 = d_of(ci)
                return (
                    pltpu.async_copy(src_hbm.at[:, pl.ds(d0, cd)],
                                     ib.at[:, pl.ds(0, cd)], sem_in[buf]),
                    pltpu.async_copy(w_hbm.at[:, pl.ds(d0, cd)],
                                     wb.at[:, pl.ds(0, cd)], sem_in[buf]),
                    pltpu.async_copy(b_hbm.at[pl.ds(d0, cd)],
                                     bb.at[pl.ds(0, cd)], sem_in[buf]),
                )

            pending_in = start_inputs(0)
            tbl_cp.wait()
            pending_out = [None] * _NBUF
            for ci in range(nch):
                buf = ci % _NBUF
                for h in pending_in:
                    h.wait()
                if ci + 1 < nch:
                    pending_in = start_inputs(ci + 1)
                ib, wb, bb, ob = bufs[buf]
                if pending_out[buf] is not None:
                    pending_out[buf].wait()

                def group(g, _, ib=ib, wb=wb, bb=bb, ob=ob):
                    col0 = pl.multiple_of(g * _L, _L)
                    bias_vec = bb[pl.ds(col0, _L)]
                    accs = [bias_vec] * _L
                    for k in range(_DEG):
                        icol = ib[k, pl.ds(col0, _L)]
                        wcol = wb[k, pl.ds(col0, _L)]
                        for j in range(_L):
                            jvec = jnp.full((_L,), j, jnp.int32)
                            accs[j] = accs[j] + wcol * plsc.load_gather(
                                table_v, [jvec, icol])
                    for j in range(_L):
                        ob[j, pl.ds(col0, _L)] = jnp.maximum(accs[j], 0.0)
                    return 0

                lax.fori_loop(0, ng, group, 0)
                pending_out[buf] = pltpu.async_copy(
                    ob.at[:, pl.ds(0, cd)], out_dst(d_of(ci), cd), sem_out[buf])
            for h in pending_out:
                if h is not None:
                    h.wait()

        run_layer(_SIZES[0], _SIZES[1], x_hbm.at[blk], s0_hbm, w0_hbm, b0_hbm,
                  lambda d0, cd: act_a.at[sblk, :, pl.ds(d0, cd)])
        plsc.subcore_barrier()
        run_layer(_SIZES[1], _SIZES[2], act_a.at[sblk], s1_hbm, w1_hbm, b1_hbm,
                  lambda d0, cd: act_b.at[blk, :, pl.ds(d0, cd)])
        plsc.subcore_barrier()
        run_layer(_SIZES[2], _SIZES[3], act_b.at[blk], s2_hbm, w2_hbm, b2_hbm,
                  lambda d0, cd: out_hbm.at[blk, :, pl.ds(d0, cd)])

    return fused_k


_FUSED = _fused()


def kernel(x, edge_index_0, weights_0, bias_0, edge_index_1, weights_1, bias_1,
           edge_index_2, weights_2, bias_2):
    xb = x.reshape(_NBLK, _L, _SIZES[0])
    cols = []
    for li, (ei, w) in enumerate(((edge_index_0, weights_0),
                                  (edge_index_1, weights_1),
                                  (edge_index_2, weights_2))):
        curr = _SIZES[li + 1]
        cols.append(ei[1].astype(jnp.int32).reshape(curr, _DEG).T)
        cols.append(w.reshape(curr, _DEG).T)
    out, _ = _FUSED(xb, cols[0], cols[1], bias_0, cols[2], cols[3], bias_1,
                    cols[4], cols[5], bias_2)
    return out.reshape(_B, _SIZES[3])
